# baseline (speedup 1.0000x reference)
import functools

import jax
import jax.numpy as jnp
from jax.experimental import pallas as pl
from jax.experimental.pallas import tpu as pltpu
from jax.experimental.pallas import tpu_sc as plsc

NUM_BINS = 256
V_MIN = -4.0
V_MAX = 4.0
BIN_W = (V_MAX - V_MIN) / NUM_BINS
INV_W = 1.0 / BIN_W
N = 33554432

SC_LANES = 16
SC_CHUNK = 8192
SC_BLOCKS = N // SC_CHUNK


def _quantize(xv):
    xc = jnp.minimum(jnp.maximum(xv, V_MIN), V_MAX)
    t = (xc - V_MIN) * INV_W
    idx = t.astype(jnp.int32)
    e_lo = idx.astype(jnp.float32) * BIN_W + V_MIN
    idx = jnp.where(xc <= e_lo, idx - 1, idx)
    e_hi = idx.astype(jnp.float32) * BIN_W + (V_MIN + BIN_W)
    idx = jnp.where(xc > e_hi, idx + 1, idx)
    idx = jnp.clip(idx, 0, NUM_BINS - 1)
    xhat = idx.astype(jnp.float32) * BIN_W + (V_MIN + 0.5 * BIN_W)
    return idx, xhat


def _sc_body(x_v, idx_v, xhat_v):
    @pl.loop(0, SC_CHUNK, step=SC_LANES)
    def _(c):
        s = (pl.ds(0, 1), pl.ds(c, SC_LANES))
        idx, xhat = _quantize(x_v.at[*s][...])
        idx_v.at[*s][...] = idx
        xhat_v.at[*s][...] = xhat


def _sc_quant(x2):
    mesh = plsc.VectorSubcoreMesh(core_axis_name="c", subcore_axis_name="s")

    @functools.partial(
        pl.kernel,
        out_type=[jax.ShapeDtypeStruct((1, N), jnp.int32),
                  jax.ShapeDtypeStruct((1, N), jnp.float32)],
        mesh=mesh,
        scratch_types=[],
    )
    def k(x_hbm, idx_hbm, xhat_hbm):
        pltpu.emit_pipeline(
            _sc_body,
            grid=(SC_BLOCKS,),
            in_specs=[pl.BlockSpec((1, SC_CHUNK), lambda i: (0, i))],
            out_specs=[pl.BlockSpec((1, SC_CHUNK), lambda i: (0, i)),
                       pl.BlockSpec((1, SC_CHUNK), lambda i: (0, i))],
            core_axis_name=("c", "s"),
            dimension_semantics=(pltpu.PARALLEL,),
        )(x_hbm, idx_hbm, xhat_hbm)

    return k(x2)


def kernel(x, edges, centres):
    idx2, xhat2 = _sc_quant(x.reshape(1, N))
    return idx2.reshape(-1), xhat2.reshape(-1)


# speedup vs baseline: 3.2606x; 3.2606x over previous
import functools

import jax
import jax.numpy as jnp
from jax.experimental import pallas as pl
from jax.experimental.pallas import tpu as pltpu
from jax.experimental.pallas import tpu_sc as plsc

NUM_BINS = 256
V_MIN = -4.0
V_MAX = 4.0
BIN_W = (V_MAX - V_MIN) / NUM_BINS
INV_W = 1.0 / BIN_W
N = 33554432

ROWS = 32768
COLS = 1024
BLOCK_ROWS = 1024

SC_LANES = 16
SC_CHUNK = 8192
SC_BLOCKS = N // SC_CHUNK


def _quant_idx(xv):
    xc = jnp.minimum(jnp.maximum(xv, V_MIN), V_MAX)
    u = (V_MAX - xc) * INV_W
    idx = (NUM_BINS - 1) - u.astype(jnp.int32)
    e_hi = idx.astype(jnp.float32) * BIN_W + (V_MIN + BIN_W)
    idx = jnp.where(xc > e_hi, idx + 1, idx)
    return jnp.maximum(idx, 0)


def _centre(idx):
    return idx.astype(jnp.float32) * BIN_W + (V_MIN + 0.5 * BIN_W)


def _tc_xhat_body(x_ref, xhat_ref):
    xhat_ref[...] = _centre(_quant_idx(x_ref[...]))


def _tc_xhat(x2):
    return pl.pallas_call(
        _tc_xhat_body,
        grid=(ROWS // BLOCK_ROWS,),
        in_specs=[pl.BlockSpec((BLOCK_ROWS, COLS), lambda i: (i, 0))],
        out_specs=pl.BlockSpec((BLOCK_ROWS, COLS), lambda i: (i, 0)),
        out_shape=jax.ShapeDtypeStruct((ROWS, COLS), jnp.float32),
        compiler_params=pltpu.CompilerParams(
            dimension_semantics=("arbitrary",)),
    )(x2)


def _sc_idx_body(x_v, idx_v):
    @plsc.parallel_loop(0, SC_CHUNK, step=SC_LANES, unroll=8)
    def _(c):
        s = (pl.ds(0, 1), pl.ds(c, SC_LANES))
        idx_v.at[*s][...] = _quant_idx(x_v.at[*s][...])


def _sc_idx(x2):
    mesh = plsc.VectorSubcoreMesh(core_axis_name="c", subcore_axis_name="s")

    @functools.partial(
        pl.kernel,
        out_type=jax.ShapeDtypeStruct((1, N), jnp.int32),
        mesh=mesh,
        scratch_types=[],
    )
    def k(x_hbm, idx_hbm):
        pltpu.emit_pipeline(
            _sc_idx_body,
            grid=(SC_BLOCKS,),
            in_specs=[pl.BlockSpec((1, SC_CHUNK), lambda i: (0, i))],
            out_specs=[pl.BlockSpec((1, SC_CHUNK), lambda i: (0, i))],
            core_axis_name=("c", "s"),
            dimension_semantics=(pltpu.PARALLEL,),
        )(x_hbm, idx_hbm)

    return k(x2)


def kernel(x, edges, centres):
    xhat2 = _tc_xhat(x.reshape(ROWS, COLS))
    idx2 = _sc_idx(x.reshape(1, N))
    return idx2.reshape(-1), xhat2.reshape(-1)


# speedup vs baseline: 3.4041x; 1.0440x over previous
import functools

import jax
import jax.numpy as jnp
from jax.experimental import pallas as pl
from jax.experimental.pallas import tpu as pltpu
from jax.experimental.pallas import tpu_sc as plsc

NUM_BINS = 256
V_MIN = -4.0
V_MAX = 4.0
BIN_W = (V_MAX - V_MIN) / NUM_BINS
HALF_W = 0.5 * BIN_W
N = 33554432

ROWS = 32768
COLS = 1024
BLOCK_ROWS = 1024

SC_LANES = 16
SC_CHUNK = 8192
SC_BLOCKS = N // SC_CHUNK

_C = 262148.0


def _round_to_grid(xv):
    xc = jnp.minimum(jnp.maximum(xv, V_MIN), V_MAX)
    r = (xc + _C) - _C
    return xc, r


def _tc_xhat_body(x_ref, xhat_ref):
    xc, r = _round_to_grid(x_ref[...])
    xhat = r + jnp.where(xc > r, HALF_W, -HALF_W).astype(jnp.float32)
    xhat_ref[...] = jnp.minimum(jnp.maximum(xhat, V_MIN + HALF_W),
                                V_MAX - HALF_W)


def _tc_xhat(x2):
    return pl.pallas_call(
        _tc_xhat_body,
        grid=(ROWS // BLOCK_ROWS,),
        in_specs=[pl.BlockSpec((BLOCK_ROWS, COLS), lambda i: (i, 0))],
        out_specs=pl.BlockSpec((BLOCK_ROWS, COLS), lambda i: (i, 0)),
        out_shape=jax.ShapeDtypeStruct((ROWS, COLS), jnp.float32),
        compiler_params=pltpu.CompilerParams(
            dimension_semantics=("arbitrary",)),
    )(x2)


def _sc_idx_body(x_v, idx_v):
    @plsc.parallel_loop(0, SC_CHUNK, step=SC_LANES, unroll=8)
    def _(c):
        s = pl.ds(c, SC_LANES)
        xc, r = _round_to_grid(x_v.at[s][...])
        m = ((r + (-V_MIN + HALF_W)) * (1.0 / BIN_W)).astype(jnp.int32)
        idx = jnp.where(xc > r, m, m - 1)
        idx_v.at[s][...] = jnp.maximum(idx, 0)


def _sc_idx(x1):
    mesh = plsc.VectorSubcoreMesh(core_axis_name="c", subcore_axis_name="s")

    @functools.partial(
        pl.kernel,
        out_type=jax.ShapeDtypeStruct((N,), jnp.int32),
        mesh=mesh,
        scratch_types=[],
    )
    def k(x_hbm, idx_hbm):
        pltpu.emit_pipeline(
            _sc_idx_body,
            grid=(SC_BLOCKS,),
            in_specs=[pl.BlockSpec((SC_CHUNK,), lambda i: (i,))],
            out_specs=[pl.BlockSpec((SC_CHUNK,), lambda i: (i,))],
            core_axis_name=("c", "s"),
            dimension_semantics=(pltpu.PARALLEL,),
        )(x_hbm, idx_hbm)

    return k(x1)


def kernel(x, edges, centres):
    xhat2 = _tc_xhat(x.reshape(ROWS, COLS))
    idx = _sc_idx(x)
    return idx, xhat2.reshape(-1)


# speedup vs baseline: 7.2394x; 2.1267x over previous
import functools

import jax
import jax.numpy as jnp
from jax.experimental import pallas as pl
from jax.experimental.pallas import tpu as pltpu
from jax.experimental.pallas import tpu_sc as plsc

NUM_BINS = 256
V_MIN = -4.0
V_MAX = 4.0
BIN_W = (V_MAX - V_MIN) / NUM_BINS
HALF_W = 0.5 * BIN_W
N = 33554432

TC_BLOCK = 1048576

SC_LANES = 16
SC_CHUNK = 8192
SC_BLOCKS = N // SC_CHUNK

_C = 262148.0


def _round_to_grid(xv):
    xc = jnp.minimum(jnp.maximum(xv, V_MIN), V_MAX)
    r = (xc + _C) - _C
    return xc, r


def _tc_xhat_body(x_ref, xhat_ref):
    xc, r = _round_to_grid(x_ref[...])
    xhat = r + jnp.where(xc > r, HALF_W, -HALF_W).astype(jnp.float32)
    xhat_ref[...] = jnp.minimum(jnp.maximum(xhat, V_MIN + HALF_W),
                                V_MAX - HALF_W)


def _tc_xhat(x1):
    return pl.pallas_call(
        _tc_xhat_body,
        grid=(N // TC_BLOCK,),
        in_specs=[pl.BlockSpec((TC_BLOCK,), lambda i: (i,))],
        out_specs=pl.BlockSpec((TC_BLOCK,), lambda i: (i,)),
        out_shape=jax.ShapeDtypeStruct((N,), jnp.float32),
        compiler_params=pltpu.CompilerParams(
            dimension_semantics=("arbitrary",)),
    )(x1)


def _sc_idx_body(x_v, idx_v):
    @plsc.parallel_loop(0, SC_CHUNK, step=SC_LANES, unroll=8)
    def _(c):
        s = pl.ds(c, SC_LANES)
        xc, r = _round_to_grid(x_v.at[s][...])
        m = ((r + (-V_MIN + HALF_W)) * (1.0 / BIN_W)).astype(jnp.int32)
        idx = jnp.where(xc > r, m, m - 1)
        idx_v.at[s][...] = jnp.maximum(idx, 0)


def _sc_idx(x1):
    mesh = plsc.VectorSubcoreMesh(core_axis_name="c", subcore_axis_name="s")

    @functools.partial(
        pl.kernel,
        out_type=jax.ShapeDtypeStruct((N,), jnp.int32),
        mesh=mesh,
        scratch_types=[],
    )
    def k(x_hbm, idx_hbm):
        pltpu.emit_pipeline(
            _sc_idx_body,
            grid=(SC_BLOCKS,),
            in_specs=[pl.BlockSpec((SC_CHUNK,), lambda i: (i,))],
            out_specs=[pl.BlockSpec((SC_CHUNK,), lambda i: (i,))],
            core_axis_name=("c", "s"),
            dimension_semantics=(pltpu.PARALLEL,),
        )(x_hbm, idx_hbm)

    return k(x1)


def kernel(x, edges, centres):
    idx = _sc_idx(x)
    xhat = _tc_xhat(x)
    return idx, xhat


# speedup vs baseline: 9.1627x; 1.2657x over previous
import functools

import jax
import jax.numpy as jnp
from jax.experimental import pallas as pl
from jax.experimental.pallas import tpu as pltpu
from jax.experimental.pallas import tpu_sc as plsc

NUM_BINS = 256
V_MIN = -4.0
V_MAX = 4.0
BIN_W = (V_MAX - V_MIN) / NUM_BINS
HALF_W = 0.5 * BIN_W
N = 33554432

TC_BLOCK = 1048576

SC_LANES = 16
SC_CHUNK = 16384
SC_BLOCKS = N // SC_CHUNK

_C = 262148.0
_CLIP_LO = V_MIN + HALF_W
_CLIP_HI = V_MAX - HALF_W


def _round_to_grid(xv):
    xc = jnp.minimum(jnp.maximum(xv, _CLIP_LO), _CLIP_HI)
    r = (xc + _C) - _C
    return xc, r


def _tc_idx_body(x_ref, idx_ref):
    xc, r = _round_to_grid(x_ref[...])
    m = ((r + (-V_MIN + HALF_W)) * (1.0 / BIN_W)).astype(jnp.int32)
    idx_ref[...] = jnp.where(xc > r, m, m - 1)


def _tc_idx(x1):
    return pl.pallas_call(
        _tc_idx_body,
        grid=(N // TC_BLOCK,),
        in_specs=[pl.BlockSpec((TC_BLOCK,), lambda i: (i,))],
        out_specs=pl.BlockSpec((TC_BLOCK,), lambda i: (i,)),
        out_shape=jax.ShapeDtypeStruct((N,), jnp.int32),
        compiler_params=pltpu.CompilerParams(
            dimension_semantics=("arbitrary",)),
    )(x1)


def _sc_xhat_body(x_v, xhat_v):
    @plsc.parallel_loop(0, SC_CHUNK, step=SC_LANES, unroll=8)
    def _(c):
        s = pl.ds(c, SC_LANES)
        xc, r = _round_to_grid(x_v.at[s][...])
        xhat_v.at[s][...] = r + jnp.where(xc > r, HALF_W, -HALF_W).astype(
            jnp.float32)


def _sc_xhat(x1):
    mesh = plsc.VectorSubcoreMesh(core_axis_name="c", subcore_axis_name="s")

    @functools.partial(
        pl.kernel,
        out_type=jax.ShapeDtypeStruct((N,), jnp.float32),
        mesh=mesh,
        scratch_types=[],
    )
    def k(x_hbm, xhat_hbm):
        pltpu.emit_pipeline(
            _sc_xhat_body,
            grid=(SC_BLOCKS,),
            in_specs=[pl.BlockSpec((SC_CHUNK,), lambda i: (i,))],
            out_specs=[pl.BlockSpec((SC_CHUNK,), lambda i: (i,))],
            core_axis_name=("c", "s"),
            dimension_semantics=(pltpu.PARALLEL,),
        )(x_hbm, xhat_hbm)

    return k(x1)


def kernel(x, edges, centres):
    xhat = _sc_xhat(x)
    idx = _tc_idx(x)
    return idx, xhat


# speedup vs baseline: 9.1860x; 1.0025x over previous
import functools

import jax
import jax.numpy as jnp
from jax.experimental import pallas as pl
from jax.experimental.pallas import tpu as pltpu
from jax.experimental.pallas import tpu_sc as plsc

NUM_BINS = 256
V_MIN = -4.0
V_MAX = 4.0
BIN_W = (V_MAX - V_MIN) / NUM_BINS
HALF_W = 0.5 * BIN_W
N = 33554432

TC_BLOCK = 1048576

SC_LANES = 16
SC_CHUNK = 16384
SC_BLOCKS = N // SC_CHUNK

_C = 262148.0
_CLIP_LO = V_MIN + HALF_W
_CLIP_HI = V_MAX - HALF_W


def _round_to_grid(xv):
    xc = jnp.minimum(jnp.maximum(xv, _CLIP_LO), _CLIP_HI)
    r = (xc + _C) - _C
    return xc, r


def _tc_idx_body(x_ref, idx_ref):
    xc, r = _round_to_grid(x_ref[...])
    m = ((r + (-V_MIN + HALF_W)) * (1.0 / BIN_W)).astype(jnp.int32)
    idx_ref[...] = jnp.where(xc > r, m, m - 1)


def _tc_idx(x1):
    return pl.pallas_call(
        _tc_idx_body,
        grid=(N // TC_BLOCK,),
        in_specs=[pl.BlockSpec((TC_BLOCK,), lambda i: (i,))],
        out_specs=pl.BlockSpec((TC_BLOCK,), lambda i: (i,)),
        out_shape=jax.ShapeDtypeStruct((N,), jnp.int32),
        compiler_params=pltpu.CompilerParams(
            dimension_semantics=("arbitrary",)),
    )(x1)


def _sc_xhat_body(x_v, xhat_v):
    @plsc.parallel_loop(0, SC_CHUNK, step=SC_LANES, unroll=16)
    def _(c):
        s = pl.ds(c, SC_LANES)
        xc, r = _round_to_grid(x_v.at[s][...])
        xhat_v.at[s][...] = r + jnp.where(xc > r, HALF_W, -HALF_W).astype(
            jnp.float32)


def _sc_xhat(x1):
    mesh = plsc.VectorSubcoreMesh(core_axis_name="c", subcore_axis_name="s")

    @functools.partial(
        pl.kernel,
        out_type=jax.ShapeDtypeStruct((N,), jnp.float32),
        mesh=mesh,
        scratch_types=[],
    )
    def k(x_hbm, xhat_hbm):
        pltpu.emit_pipeline(
            _sc_xhat_body,
            grid=(SC_BLOCKS,),
            in_specs=[pl.BlockSpec((SC_CHUNK,), lambda i: (i,))],
            out_specs=[pl.BlockSpec((SC_CHUNK,), lambda i: (i,))],
            core_axis_name=("c", "s"),
            dimension_semantics=(pltpu.PARALLEL,),
        )(x_hbm, xhat_hbm)

    return k(x1)


def kernel(x, edges, centres):
    xhat = _sc_xhat(x)
    idx = _tc_idx(x)
    return idx, xhat


# speedup vs baseline: 9.2102x; 1.0026x over previous
import functools

import jax
import jax.numpy as jnp
from jax.experimental import pallas as pl
from jax.experimental.pallas import tpu as pltpu
from jax.experimental.pallas import tpu_sc as plsc

NUM_BINS = 256
V_MIN = -4.0
V_MAX = 4.0
BIN_W = (V_MAX - V_MIN) / NUM_BINS
HALF_W = 0.5 * BIN_W
N = 33554432

TC_BLOCK = 1048576

SC_LANES = 16
SC_CHUNK = 16384
SC_BLOCKS = N // SC_CHUNK

_C = 262148.0
_CLIP_LO = V_MIN + HALF_W
_CLIP_HI = V_MAX - HALF_W


def _round_to_grid(xv):
    xc = jnp.minimum(jnp.maximum(xv, _CLIP_LO), _CLIP_HI)
    r = (xc + _C) - _C
    return xc, r


def _tc_idx_body(x_ref, idx_ref):
    xc, r = _round_to_grid(x_ref[...])
    m = ((r + (-V_MIN + HALF_W)) * (1.0 / BIN_W)).astype(jnp.int32)
    idx_ref[...] = jnp.where(xc > r, m, m - 1)


def _tc_idx(x1):
    return pl.pallas_call(
        _tc_idx_body,
        grid=(N // TC_BLOCK,),
        in_specs=[pl.BlockSpec((TC_BLOCK,), lambda i: (i,))],
        out_specs=pl.BlockSpec((TC_BLOCK,), lambda i: (i,)),
        out_shape=jax.ShapeDtypeStruct((N,), jnp.int32),
        compiler_params=pltpu.CompilerParams(
            dimension_semantics=("arbitrary",)),
    )(x1)


SC_WORKERS = 32
SC_SHARD = N // SC_WORKERS
SC_NCHUNK = SC_SHARD // SC_CHUNK


def _sc_compute(x_v, xhat_v):
    @plsc.parallel_loop(0, SC_CHUNK, step=SC_LANES, unroll=8)
    def _(c):
        s = pl.ds(c, SC_LANES)
        xc, r = _round_to_grid(x_v.at[s][...])
        xhat_v.at[s][...] = r + jnp.where(xc > r, HALF_W, -HALF_W).astype(
            jnp.float32)


def _sc_xhat(x1):
    mesh = plsc.VectorSubcoreMesh(core_axis_name="c", subcore_axis_name="s")

    @functools.partial(
        pl.kernel,
        out_type=jax.ShapeDtypeStruct((N,), jnp.float32),
        mesh=mesh,
        scratch_types=[
            pltpu.VMEM((SC_CHUNK,), jnp.float32),
            pltpu.VMEM((SC_CHUNK,), jnp.float32),
            pltpu.VMEM((SC_CHUNK,), jnp.float32),
            pltpu.VMEM((SC_CHUNK,), jnp.float32),
            pltpu.SemaphoreType.DMA,
            pltpu.SemaphoreType.DMA,
            pltpu.SemaphoreType.DMA,
            pltpu.SemaphoreType.DMA,
        ],
    )
    def k(x_hbm, xhat_hbm, in0, in1, out0, out1, si0, si1, so0, so1):
        wid = jax.lax.axis_index("s") * 2 + jax.lax.axis_index("c")
        base = wid * SC_SHARD
        ins, outs = (in0, in1), (out0, out1)
        sis, sos = (si0, si1), (so0, so1)
        pltpu.async_copy(x_hbm.at[pl.ds(base, SC_CHUNK)], in0, si0)
        pltpu.async_copy(x_hbm.at[pl.ds(base + SC_CHUNK, SC_CHUNK)], in1, si1)

        @pl.loop(0, SC_NCHUNK, step=2)
        def _(g):
            for b in range(2):
                j = g + b
                off = base + j * SC_CHUNK
                pltpu.make_async_copy(
                    x_hbm.at[pl.ds(off, SC_CHUNK)], ins[b], sis[b]).wait()

                @pl.when(j >= 2)
                def _():
                    pltpu.make_async_copy(
                        outs[b],
                        xhat_hbm.at[pl.ds(off - 2 * SC_CHUNK, SC_CHUNK)],
                        sos[b]).wait()

                _sc_compute(ins[b], outs[b])
                pltpu.async_copy(
                    outs[b], xhat_hbm.at[pl.ds(off, SC_CHUNK)], sos[b])

                @pl.when(j + 2 < SC_NCHUNK)
                def _():
                    pltpu.async_copy(
                        x_hbm.at[pl.ds(off + 2 * SC_CHUNK, SC_CHUNK)],
                        ins[b], sis[b])

        pltpu.make_async_copy(
            out0, xhat_hbm.at[pl.ds(base + (SC_NCHUNK - 2) * SC_CHUNK,
                                    SC_CHUNK)], so0).wait()
        pltpu.make_async_copy(
            out1, xhat_hbm.at[pl.ds(base + (SC_NCHUNK - 1) * SC_CHUNK,
                                    SC_CHUNK)], so1).wait()

    return k(x1)


def kernel(x, edges, centres):
    xhat = _sc_xhat(x)
    idx = _tc_idx(x)
    return idx, xhat
